# baseline (speedup 1.0000x reference)
import functools

import jax
import jax.numpy as jnp
from jax import lax
from jax.experimental import pallas as pl
from jax.experimental.pallas import tpu as pltpu
from jax.experimental.pallas import tpu_sc as plsc

_B = 16384
_WIDE = 1000
_NCAT = 26
_NCONT = 13
_VOCAB = 100000
_EDIM = 16

_NC = 2
_NS = 16
_NW = _NC * _NS

_T = _NCAT * _EDIM
_TPW = _T // _NW
_IC = 2048
_NIC = _B // _IC


def _sc_gather_body(table_hbm, idx_hbm, out_hbm, row_v, idx_c0, idx_c1,
                    ostage, sem, isem):
    wid = lax.axis_index("s") * _NC + lax.axis_index("c")
    idx_cs = (idx_c0, idx_c1)

    def row_body(r, _):
        t = wid * _TPW + r
        j = t // _EDIM
        pltpu.sync_copy(table_hbm.at[t], row_v)
        pltpu.sync_copy(idx_hbm.at[j, pl.ds(0, _IC)], idx_c0)
        for cb in range(_NIC):
            idx_c = idx_cs[cb % 2]
            cp = None
            if cb + 1 < _NIC:
                cp = pltpu.async_copy(
                    idx_hbm.at[j, pl.ds((cb + 1) * _IC, _IC)],
                    idx_cs[(cb + 1) % 2], isem)

            def gather_body(g, _, idx_c=idx_c, cb=cb):
                idxv = idx_c[pl.ds(g * 16, 16)]
                ostage[pl.ds(cb * _IC + g * 16, 16)] = plsc.load_gather(
                    row_v, [idxv])
                return 0
            lax.fori_loop(0, _IC // 16, gather_body, 0, unroll=16)
            if cp is not None:
                cp.wait()
        pltpu.async_copy(ostage, out_hbm.at[t], sem).wait()
        return 0
    lax.fori_loop(0, _TPW, row_body, 0)


@functools.cache
def _sc_gather():
    return functools.partial(
        pl.kernel,
        out_type=jax.ShapeDtypeStruct((_T, _B), jnp.float32),
        mesh=plsc.VectorSubcoreMesh(core_axis_name="c", subcore_axis_name="s"),
        compiler_params=pltpu.CompilerParams(needs_layout_passes=False),
        scratch_types=[
            pltpu.VMEM((_VOCAB,), jnp.float32),
            pltpu.VMEM((_IC,), jnp.int32),
            pltpu.VMEM((_IC,), jnp.int32),
            pltpu.VMEM((_B,), jnp.float32),
            pltpu.SemaphoreType.DMA,
            pltpu.SemaphoreType.DMA,
        ],
    )(_sc_gather_body)


_TB = 4096


def _wide_body(xw_ref, wow_ref, out_ref):
    out_ref[...] = jnp.dot(xw_ref[...], wow_ref[...],
                           preferred_element_type=jnp.float32)


def _wide_call(X_w, Wo_w):
    grid = _B // _TB
    return pl.pallas_call(
        _wide_body,
        grid=(grid,),
        in_specs=[
            pl.BlockSpec((_TB, _WIDE), lambda i: (i, 0)),
            pl.BlockSpec((_WIDE, 1), lambda i: (0, 0)),
        ],
        out_specs=pl.BlockSpec((_TB, 1), lambda i: (i, 0)),
        out_shape=jax.ShapeDtypeStruct((_B, 1), jnp.float32),
        compiler_params=pltpu.CompilerParams(
            dimension_semantics=("arbitrary",)),
    )(X_w, Wo_w)


def _mlp_body(deepT_ref, cont_ref, wide_ref, w1a_ref, w1b_ref, b1_ref,
              w2_ref, b2_ref, w3_ref, b3_ref, wod_ref, bo_ref,
              out_ref):
    x = lax.dot_general(deepT_ref[...], w1a_ref[...],
                        (((0,), (0,)), ((), ())),
                        preferred_element_type=jnp.float32)
    x = x + jnp.dot(cont_ref[...], w1b_ref[...],
                    preferred_element_type=jnp.float32)
    x = jax.nn.relu(x + b1_ref[...])
    x = jax.nn.relu(jnp.dot(x, w2_ref[...],
                            preferred_element_type=jnp.float32) + b2_ref[...])
    x = jax.nn.relu(jnp.dot(x, w3_ref[...],
                            preferred_element_type=jnp.float32) + b3_ref[...])
    acc = jnp.dot(x, wod_ref[...], preferred_element_type=jnp.float32)
    out_ref[...] = jax.nn.sigmoid(acc + wide_ref[...] + bo_ref[...])


def _mlp_call(deepT, cont, wide, W1a, W1b, b1, W2, b2, W3, b3, Wo_d, bo):
    h1, h2, h3 = 256, 128, 64
    grid = _B // _TB
    full = lambda shape: pl.BlockSpec(shape, lambda i: (0,) * len(shape))
    return pl.pallas_call(
        _mlp_body,
        grid=(grid,),
        in_specs=[
            pl.BlockSpec((_T, _TB), lambda i: (0, i)),
            pl.BlockSpec((_TB, _NCONT), lambda i: (i, 0)),
            pl.BlockSpec((_TB, 1), lambda i: (i, 0)),
            full((_T, h1)),
            full((_NCONT, h1)),
            full((1, h1)),
            full((h1, h2)),
            full((1, h2)),
            full((h2, h3)),
            full((1, h3)),
            full((h3, 1)),
            full((1, 1)),
        ],
        out_specs=pl.BlockSpec((_TB, 1), lambda i: (i, 0)),
        out_shape=jax.ShapeDtypeStruct((_B, 1), jnp.float32),
        compiler_params=pltpu.CompilerParams(
            dimension_semantics=("arbitrary",)),
    )(deepT, cont, wide, W1a, W1b, b1, W2, b2, W3, b3, Wo_d, bo)


@jax.jit
def kernel(X_w, X_d, emb, W1, b1, W2, b2, W3, b3, Wo, bo):
    table_t = emb.transpose(0, 2, 1).reshape(_T, _VOCAB)
    idx_t = X_d[:, :_NCAT].T
    deepT = _sc_gather()(table_t, idx_t)
    wide = _wide_call(X_w, Wo[64:])
    cont = X_d[:, _NCAT:].astype(jnp.float32)
    out = _mlp_call(
        deepT, cont, wide,
        W1[:_T], W1[_T:],
        b1.reshape(1, -1), W2, b2.reshape(1, -1), W3, b3.reshape(1, -1),
        Wo[:64], bo.reshape(1, 1))
    return out


# speedup vs baseline: 1.0025x; 1.0025x over previous
import functools

import jax
import jax.numpy as jnp
from jax import lax
from jax.experimental import pallas as pl
from jax.experimental.pallas import tpu as pltpu
from jax.experimental.pallas import tpu_sc as plsc

_B = 16384
_WIDE = 1000
_NCAT = 26
_NCONT = 13
_VOCAB = 100000
_EDIM = 16

_NC = 2
_NS = 16
_NW = _NC * _NS

_T = _NCAT * _EDIM
_TPW = _T // _NW
_IC = 2048
_NIC = _B // _IC


def _sc_gather_body(table_hbm, idx_hbm, out_hbm, row_v, idx_c0, idx_c1,
                    ostage, sem, isem):
    wid = lax.axis_index("s") * _NC + lax.axis_index("c")
    idx_cs = (idx_c0, idx_c1)

    def row_body(r, _):
        t = wid * _TPW + r
        j = t // _EDIM
        pltpu.sync_copy(table_hbm.at[t], row_v)
        pltpu.sync_copy(idx_hbm.at[j, pl.ds(0, _IC)], idx_c0)
        for cb in range(_NIC):
            idx_c = idx_cs[cb % 2]
            cp = None
            if cb + 1 < _NIC:
                cp = pltpu.async_copy(
                    idx_hbm.at[j, pl.ds((cb + 1) * _IC, _IC)],
                    idx_cs[(cb + 1) % 2], isem)

            def gather_body(g, _, idx_c=idx_c, cb=cb):
                idxv = idx_c[pl.ds(g * 16, 16)]
                ostage[pl.ds(cb * _IC + g * 16, 16)] = plsc.load_gather(
                    row_v, [idxv])
                return 0
            lax.fori_loop(0, _IC // 16, gather_body, 0, unroll=8)
            if cp is not None:
                cp.wait()
        pltpu.async_copy(ostage, out_hbm.at[t], sem).wait()
        return 0
    lax.fori_loop(0, _TPW, row_body, 0)


@functools.cache
def _sc_gather():
    return functools.partial(
        pl.kernel,
        out_type=jax.ShapeDtypeStruct((_T, _B), jnp.float32),
        mesh=plsc.VectorSubcoreMesh(core_axis_name="c", subcore_axis_name="s"),
        compiler_params=pltpu.CompilerParams(needs_layout_passes=False),
        scratch_types=[
            pltpu.VMEM((_VOCAB,), jnp.float32),
            pltpu.VMEM((_IC,), jnp.int32),
            pltpu.VMEM((_IC,), jnp.int32),
            pltpu.VMEM((_B,), jnp.float32),
            pltpu.SemaphoreType.DMA,
            pltpu.SemaphoreType.DMA,
        ],
    )(_sc_gather_body)


_TB = 2048


def _wide_body(xw_ref, wow_ref, out_ref):
    out_ref[...] = jnp.dot(xw_ref[...], wow_ref[...],
                           preferred_element_type=jnp.float32)


def _wide_call(X_w, Wo_w):
    grid = _B // _TB
    return pl.pallas_call(
        _wide_body,
        grid=(grid,),
        in_specs=[
            pl.BlockSpec((_TB, _WIDE), lambda i: (i, 0)),
            pl.BlockSpec((_WIDE, 1), lambda i: (0, 0)),
        ],
        out_specs=pl.BlockSpec((_TB, 1), lambda i: (i, 0)),
        out_shape=jax.ShapeDtypeStruct((_B, 1), jnp.float32),
        compiler_params=pltpu.CompilerParams(
            dimension_semantics=("arbitrary",)),
    )(X_w, Wo_w)


def _mlp_body(deepT_ref, cont_ref, wide_ref, w1a_ref, w1b_ref, b1_ref,
              w2_ref, b2_ref, w3_ref, b3_ref, wod_ref, bo_ref,
              out_ref):
    x = lax.dot_general(deepT_ref[...], w1a_ref[...],
                        (((0,), (0,)), ((), ())),
                        preferred_element_type=jnp.float32)
    x = x + jnp.dot(cont_ref[...], w1b_ref[...],
                    preferred_element_type=jnp.float32)
    x = jax.nn.relu(x + b1_ref[...])
    x = jax.nn.relu(jnp.dot(x, w2_ref[...],
                            preferred_element_type=jnp.float32) + b2_ref[...])
    x = jax.nn.relu(jnp.dot(x, w3_ref[...],
                            preferred_element_type=jnp.float32) + b3_ref[...])
    acc = jnp.dot(x, wod_ref[...], preferred_element_type=jnp.float32)
    out_ref[...] = jax.nn.sigmoid(acc + wide_ref[...] + bo_ref[...])


def _mlp_call(deepT, cont, wide, W1a, W1b, b1, W2, b2, W3, b3, Wo_d, bo):
    h1, h2, h3 = 256, 128, 64
    grid = _B // _TB
    full = lambda shape: pl.BlockSpec(shape, lambda i: (0,) * len(shape))
    return pl.pallas_call(
        _mlp_body,
        grid=(grid,),
        in_specs=[
            pl.BlockSpec((_T, _TB), lambda i: (0, i)),
            pl.BlockSpec((_TB, _NCONT), lambda i: (i, 0)),
            pl.BlockSpec((_TB, 1), lambda i: (i, 0)),
            full((_T, h1)),
            full((_NCONT, h1)),
            full((1, h1)),
            full((h1, h2)),
            full((1, h2)),
            full((h2, h3)),
            full((1, h3)),
            full((h3, 1)),
            full((1, 1)),
        ],
        out_specs=pl.BlockSpec((_TB, 1), lambda i: (i, 0)),
        out_shape=jax.ShapeDtypeStruct((_B, 1), jnp.float32),
        compiler_params=pltpu.CompilerParams(
            dimension_semantics=("arbitrary",)),
    )(deepT, cont, wide, W1a, W1b, b1, W2, b2, W3, b3, Wo_d, bo)


@jax.jit
def kernel(X_w, X_d, emb, W1, b1, W2, b2, W3, b3, Wo, bo):
    table_t = emb.transpose(0, 2, 1).reshape(_T, _VOCAB)
    idx_t = X_d[:, :_NCAT].T
    deepT = _sc_gather()(table_t, idx_t)
    wide = _wide_call(X_w, Wo[64:])
    cont = X_d[:, _NCAT:].astype(jnp.float32)
    out = _mlp_call(
        deepT, cont, wide,
        W1[:_T], W1[_T:],
        b1.reshape(1, -1), W2, b2.reshape(1, -1), W3, b3.reshape(1, -1),
        Wo[:64], bo.reshape(1, 1))
    return out
